# Initial kernel scaffold; baseline (speedup 1.0000x reference)
import jax
import jax.numpy as jnp
from jax.experimental import pallas as pl


def kernel(x, t_feature_edge, t_edge_index, Wx1, bx1, Wx2, bx2, We1, be1, Wl, bl, Wr, br, We2, be2, Wf, bf):
    raise NotImplementedError("write your pallas kernel here")


# speedup vs baseline: 1.6624x; 1.6624x over previous
import functools

import jax
import jax.numpy as jnp
from jax import lax
from jax.experimental import pallas as pl
from jax.experimental.pallas import tpu as pltpu
from jax.experimental.pallas import tpu_sc as plsc

N = 10000
E = 320000
D = 128
DE = 16

NC = 2
NS = 16
NW = NC * NS
EPW = E // NW
CHUNK = 48
NCHUNK = EPW // CHUNK
TAILC = EPW - NCHUNK * CHUNK
NPAD = 10240
RPW = NPAD // NS
RZ = 128


def _head_body(x_ref, w1_ref, b1_ref, w2_ref, b2_ref, o1_ref, o2_ref):
    xb = x_ref[...]
    o1_ref[...] = jnp.maximum(
        jnp.dot(xb, w1_ref[...], preferred_element_type=jnp.float32) + b1_ref[...], 0.0)
    o2_ref[...] = jnp.maximum(
        jnp.dot(xb, w2_ref[...], preferred_element_type=jnp.float32) + b2_ref[...], 0.0)


def _head(x, Wx1, bx1, Wx2, bx2):
    nb = 10
    blk = N // nb
    return pl.pallas_call(
        _head_body,
        grid=(nb,),
        in_specs=[
            pl.BlockSpec((blk, D), lambda i: (i, 0)),
            pl.BlockSpec((D, D), lambda i: (0, 0)),
            pl.BlockSpec((1, D), lambda i: (0, 0)),
            pl.BlockSpec((D, D), lambda i: (0, 0)),
            pl.BlockSpec((1, D), lambda i: (0, 0)),
        ],
        out_specs=[
            pl.BlockSpec((blk, D), lambda i: (i, 0)),
            pl.BlockSpec((blk, D), lambda i: (i, 0)),
        ],
        out_shape=[
            jax.ShapeDtypeStruct((N, D), jnp.float32),
            jax.ShapeDtypeStruct((N, D), jnp.float32),
        ],
    )(x, Wx1, bx1.reshape(1, D), Wx2, bx2.reshape(1, D))


def _edge_body(fe_ref, w_ref, b_ref, o_ref):
    o_ref[...] = jnp.dot(fe_ref[...], w_ref[...],
                         preferred_element_type=jnp.float32) + b_ref[...]


def _edge_linear(t_feature_edge, We1, be1):
    nb = 80
    blk = E // nb
    return pl.pallas_call(
        _edge_body,
        grid=(nb,),
        in_specs=[
            pl.BlockSpec((blk, DE), lambda i: (i, 0)),
            pl.BlockSpec((DE, D), lambda i: (0, 0)),
            pl.BlockSpec((1, D), lambda i: (0, 0)),
        ],
        out_specs=pl.BlockSpec((blk, D), lambda i: (i, 0)),
        out_shape=jax.ShapeDtypeStruct((E, D), jnp.float32),
    )(t_feature_edge, We1, be1.reshape(1, D))


def _tail_body(a0_ref, a1_ref, c0_ref, c1_ref, x1_ref, x2_ref,
               wl_ref, bl_ref, wr_ref, br_ref, we2_ref, be2_ref,
               wf_ref, bf_ref, o_ref):
    deg = jnp.maximum(c0_ref[...][:, :1] + c1_ref[...][:, :1], 1.0)
    agg = (a0_ref[...] + a1_ref[...]) / deg
    t = (jnp.dot(agg, wl_ref[...], preferred_element_type=jnp.float32) + bl_ref[...]
         + jnp.dot(x1_ref[...], wr_ref[...], preferred_element_type=jnp.float32)
         + br_ref[...])
    h1 = jnp.maximum(
        jnp.dot(t, we2_ref[...], preferred_element_type=jnp.float32) + be2_ref[...], 0.0)
    o_ref[...] = (jnp.dot(h1 + x2_ref[...], wf_ref[...],
                          preferred_element_type=jnp.float32) + bf_ref[...])


def _tail(a0, a1, c0, c1, x1, x2, Wl, bl, Wr, br, We2, be2, Wf, bf):
    nb = 10
    blk = N // nb
    row = lambda i: (i, 0)
    full = lambda i: (0, 0)
    return pl.pallas_call(
        _tail_body,
        grid=(nb,),
        in_specs=[
            pl.BlockSpec((blk, D), row),
            pl.BlockSpec((blk, D), row),
            pl.BlockSpec((blk, D), row),
            pl.BlockSpec((blk, D), row),
            pl.BlockSpec((blk, D), row),
            pl.BlockSpec((blk, D), row),
            pl.BlockSpec((D, D), full),
            pl.BlockSpec((1, D), full),
            pl.BlockSpec((D, D), full),
            pl.BlockSpec((1, D), full),
            pl.BlockSpec((D, D), full),
            pl.BlockSpec((1, D), full),
            pl.BlockSpec((D, D), full),
            pl.BlockSpec((1, D), full),
        ],
        out_specs=pl.BlockSpec((blk, D), row),
        out_shape=jax.ShapeDtypeStruct((N, D), jnp.float32),
    )(a0, a1, c0, c1, x1, x2, Wl, bl.reshape(1, D), Wr, br.reshape(1, D),
      We2, be2.reshape(1, D), Wf, bf.reshape(1, D))


def _sc_body(f1_hbm, src_hbm, dst_hbm, x1_hbm, agg_out, cnt_out,
             agg_sh, f1_v, xg_v, src_v, dst_v, st_v, dt_v, sem):
    cid = lax.axis_index("c")
    sid = lax.axis_index("s")
    wid = sid * NC + cid

    zero16 = jnp.zeros((16,), jnp.float32)
    one16 = jnp.ones((16,), jnp.float32)

    def fill_f1(val16):
        def fill(i, _):
            r = i // (D // 16)
            k = (i % (D // 16)) * 16
            f1_v[r, pl.ds(k, 16)] = val16
            return 0
        lax.fori_loop(0, CHUNK * (D // 16), fill, 0)

    def zero_acc():
        for b in range(RPW // CHUNK):
            r0 = sid * RPW + b * CHUNK
            pltpu.sync_copy(f1_v, agg_sh.at[pl.ds(r0, CHUNK)])
        rem = RPW - (RPW // CHUNK) * CHUNK
        if rem:
            r0 = sid * RPW + (RPW // CHUNK) * CHUNK
            pltpu.sync_copy(f1_v.at[pl.ds(0, rem)], agg_sh.at[pl.ds(r0, rem)])

    def write_acc(out):
        for b in range(RPW // RZ):
            r0 = sid * RPW + b * RZ
            pltpu.sync_copy(agg_sh.at[pl.ds(r0, RZ)], out.at[cid, pl.ds(r0, RZ)])

    ebase = wid * EPW

    def do_chunk(b, n, sv, dv, ones_mode):
        pltpu.sync_copy(dst_hbm.at[pl.ds(b, n)], dv)
        if not ones_mode:
            pltpu.sync_copy(src_hbm.at[pl.ds(b, n)], sv)
            pltpu.sync_copy(f1_hbm.at[pl.ds(b, n)], f1_v.at[pl.ds(0, n)])
            pltpu.async_copy(x1_hbm.at[sv], xg_v.at[pl.ds(0, n)], sem).wait()

            def mul_body(i, _):
                r = i // (D // 16)
                k = (i % (D // 16)) * 16
                f1_v[r, pl.ds(k, 16)] = f1_v[r, pl.ds(k, 16)] * xg_v[r, pl.ds(k, 16)]
                return 0
            lax.fori_loop(0, n * (D // 16), mul_body, 0)

        pltpu.sync_copy(f1_v.at[pl.ds(0, n)], agg_sh.at[dv], add=True)

    def edge_pass(ones_mode, out):
        def chunk_body(t, _):
            do_chunk(ebase + t * CHUNK, CHUNK, src_v, dst_v, ones_mode)
            return 0
        lax.fori_loop(0, NCHUNK, chunk_body, 0)
        if TAILC:
            do_chunk(ebase + NCHUNK * CHUNK, TAILC, st_v, dt_v, ones_mode)
        plsc.subcore_barrier()
        write_acc(out)
        plsc.subcore_barrier()

    fill_f1(zero16)
    zero_acc()
    plsc.subcore_barrier()
    edge_pass(False, agg_out)

    fill_f1(zero16)
    zero_acc()
    fill_f1(one16)
    plsc.subcore_barrier()
    edge_pass(True, cnt_out)


@functools.cache
def _sc_edge_kernel():
    return pl.kernel(
        _sc_body,
        out_type=(
            jax.ShapeDtypeStruct((NC, NPAD, D), jnp.float32),
            jax.ShapeDtypeStruct((NC, NPAD, D), jnp.float32),
        ),
        mesh=plsc.VectorSubcoreMesh(core_axis_name="c", subcore_axis_name="s",
                                    num_cores=NC, num_subcores=NS),
        scratch_types=[
            pltpu.VMEM_SHARED((NPAD, D), jnp.float32),
            pltpu.VMEM((CHUNK, D), jnp.float32),
            pltpu.VMEM((CHUNK, D), jnp.float32),
            pltpu.VMEM((CHUNK,), jnp.int32),
            pltpu.VMEM((CHUNK,), jnp.int32),
            pltpu.VMEM((TAILC,), jnp.int32),
            pltpu.VMEM((TAILC,), jnp.int32),
            pltpu.SemaphoreType.DMA,
        ],
    )


def kernel(x, t_feature_edge, t_edge_index, Wx1, bx1, Wx2, bx2,
           We1, be1, Wl, bl, Wr, br, We2, be2, Wf, bf):
    x_lin_1, x_lin_2 = _head(x, Wx1, bx1, Wx2, bx2)
    feature1 = _edge_linear(t_feature_edge, We1, be1)
    src = t_edge_index[0]
    dst = t_edge_index[1]
    agg_p, cnt_p = _sc_edge_kernel()(feature1, src, dst, x_lin_1)
    return _tail(agg_p[0], agg_p[1], cnt_p[0], cnt_p[1], x_lin_1, x_lin_2,
                 Wl, bl, Wr, br, We2, be2, Wf, bf)


# speedup vs baseline: 2.5336x; 1.5241x over previous
import functools

import jax
import jax.numpy as jnp
from jax import lax
from jax.experimental import pallas as pl
from jax.experimental.pallas import tpu as pltpu
from jax.experimental.pallas import tpu_sc as plsc

N = 10000
E = 320000
D = 128
DE = 16

NC = 2
NS = 16
NW = NC * NS
EPW = E // NW
CHUNK = 80
NCHUNK = EPW // CHUNK
NPAD = 10240
RPW = NPAD // NS
RZ = 128


def _head_body(x_ref, w1_ref, b1_ref, w2_ref, b2_ref, o1_ref, o2_ref):
    xb = x_ref[...]
    o1_ref[...] = jnp.maximum(
        jnp.dot(xb, w1_ref[...], preferred_element_type=jnp.float32) + b1_ref[...], 0.0)
    o2_ref[...] = jnp.maximum(
        jnp.dot(xb, w2_ref[...], preferred_element_type=jnp.float32) + b2_ref[...], 0.0)


def _head(x, Wx1, bx1, Wx2, bx2):
    nb = 10
    blk = N // nb
    return pl.pallas_call(
        _head_body,
        grid=(nb,),
        in_specs=[
            pl.BlockSpec((blk, D), lambda i: (i, 0)),
            pl.BlockSpec((D, D), lambda i: (0, 0)),
            pl.BlockSpec((1, D), lambda i: (0, 0)),
            pl.BlockSpec((D, D), lambda i: (0, 0)),
            pl.BlockSpec((1, D), lambda i: (0, 0)),
        ],
        out_specs=[
            pl.BlockSpec((blk, D), lambda i: (i, 0)),
            pl.BlockSpec((blk, D), lambda i: (i, 0)),
        ],
        out_shape=[
            jax.ShapeDtypeStruct((N, D), jnp.float32),
            jax.ShapeDtypeStruct((N, D), jnp.float32),
        ],
    )(x, Wx1, bx1.reshape(1, D), Wx2, bx2.reshape(1, D))


def _edge_body(fe_ref, w_ref, b_ref, o_ref):
    o_ref[...] = jnp.dot(fe_ref[...], w_ref[...],
                         preferred_element_type=jnp.float32) + b_ref[...]


def _edge_linear(t_feature_edge, We1, be1):
    nb = 80
    blk = E // nb
    return pl.pallas_call(
        _edge_body,
        grid=(nb,),
        in_specs=[
            pl.BlockSpec((blk, DE), lambda i: (i, 0)),
            pl.BlockSpec((DE, D), lambda i: (0, 0)),
            pl.BlockSpec((1, D), lambda i: (0, 0)),
        ],
        out_specs=pl.BlockSpec((blk, D), lambda i: (i, 0)),
        out_shape=jax.ShapeDtypeStruct((E, D), jnp.float32),
    )(t_feature_edge, We1, be1.reshape(1, D))


def _tail_body(a0_ref, a1_ref, c0_ref, c1_ref, x1_ref, x2_ref,
               wl_ref, bl_ref, wr_ref, br_ref, we2_ref, be2_ref,
               wf_ref, bf_ref, o_ref):
    deg = jnp.maximum(c0_ref[...][:, :1] + c1_ref[...][:, :1], 1.0)
    agg = (a0_ref[...] + a1_ref[...]) / deg
    t = (jnp.dot(agg, wl_ref[...], preferred_element_type=jnp.float32) + bl_ref[...]
         + jnp.dot(x1_ref[...], wr_ref[...], preferred_element_type=jnp.float32)
         + br_ref[...])
    h1 = jnp.maximum(
        jnp.dot(t, we2_ref[...], preferred_element_type=jnp.float32) + be2_ref[...], 0.0)
    o_ref[...] = (jnp.dot(h1 + x2_ref[...], wf_ref[...],
                          preferred_element_type=jnp.float32) + bf_ref[...])


def _tail(a0, a1, c0, c1, x1, x2, Wl, bl, Wr, br, We2, be2, Wf, bf):
    nb = 10
    blk = N // nb
    row = lambda i: (i, 0)
    full = lambda i: (0, 0)
    return pl.pallas_call(
        _tail_body,
        grid=(nb,),
        in_specs=[
            pl.BlockSpec((blk, D), row),
            pl.BlockSpec((blk, D), row),
            pl.BlockSpec((blk, D), row),
            pl.BlockSpec((blk, D), row),
            pl.BlockSpec((blk, D), row),
            pl.BlockSpec((blk, D), row),
            pl.BlockSpec((D, D), full),
            pl.BlockSpec((1, D), full),
            pl.BlockSpec((D, D), full),
            pl.BlockSpec((1, D), full),
            pl.BlockSpec((D, D), full),
            pl.BlockSpec((1, D), full),
            pl.BlockSpec((D, D), full),
            pl.BlockSpec((1, D), full),
        ],
        out_specs=pl.BlockSpec((blk, D), row),
        out_shape=jax.ShapeDtypeStruct((N, D), jnp.float32),
    )(a0, a1, c0, c1, x1, x2, Wl, bl.reshape(1, D), Wr, br.reshape(1, D),
      We2, be2.reshape(1, D), Wf, bf.reshape(1, D))


def _sc_body(f1_hbm, src_hbm, dst_hbm, x1_hbm, agg_out, cnt_out,
             agg_sh, f1_v, xg_v, src_v, dst_v, sem):
    cid = lax.axis_index("c")
    sid = lax.axis_index("s")
    wid = sid * NC + cid

    zero16 = jnp.zeros((16,), jnp.float32)
    one16 = jnp.ones((16,), jnp.float32)

    def fill_f1(val16):
        def fill(i, _):
            r = i // (D // 16)
            k = (i % (D // 16)) * 16
            f1_v[r, pl.ds(k, 16)] = val16
            return 0
        lax.fori_loop(0, CHUNK * (D // 16), fill, 0)

    def zero_acc():
        for b in range(RPW // CHUNK):
            r0 = sid * RPW + b * CHUNK
            pltpu.sync_copy(f1_v, agg_sh.at[pl.ds(r0, CHUNK)])
        rem = RPW - (RPW // CHUNK) * CHUNK
        if rem:
            r0 = sid * RPW + (RPW // CHUNK) * CHUNK
            pltpu.sync_copy(f1_v.at[pl.ds(0, rem)], agg_sh.at[pl.ds(r0, rem)])

    def write_acc(out):
        for b in range(RPW // RZ):
            r0 = sid * RPW + b * RZ
            pltpu.sync_copy(agg_sh.at[pl.ds(r0, RZ)], out.at[cid, pl.ds(r0, RZ)])

    ebase = wid * EPW

    def do_chunk(b, ones_mode):
        pltpu.sync_copy(dst_hbm.at[pl.ds(b, CHUNK)], dst_v)
        if not ones_mode:
            pltpu.sync_copy(src_hbm.at[pl.ds(b, CHUNK)], src_v)
            pltpu.sync_copy(f1_hbm.at[pl.ds(b, CHUNK)], f1_v)
            pltpu.async_copy(x1_hbm.at[src_v], xg_v, sem).wait()

            def mul_body(r, _):
                for k in range(D // 16):
                    f1_v[r, pl.ds(k * 16, 16)] = (
                        f1_v[r, pl.ds(k * 16, 16)] * xg_v[r, pl.ds(k * 16, 16)])
                return 0
            lax.fori_loop(0, CHUNK, mul_body, 0)

        pltpu.sync_copy(f1_v, agg_sh.at[dst_v], add=True)

    def edge_pass(ones_mode, out):
        def chunk_body(t, _):
            do_chunk(ebase + t * CHUNK, ones_mode)
            return 0
        lax.fori_loop(0, NCHUNK, chunk_body, 0)
        plsc.subcore_barrier()
        write_acc(out)
        plsc.subcore_barrier()

    fill_f1(zero16)
    zero_acc()
    plsc.subcore_barrier()
    edge_pass(False, agg_out)

    fill_f1(zero16)
    zero_acc()
    fill_f1(one16)
    plsc.subcore_barrier()
    edge_pass(True, cnt_out)


@functools.cache
def _sc_edge_kernel():
    return pl.kernel(
        _sc_body,
        out_type=(
            jax.ShapeDtypeStruct((NC, NPAD, D), jnp.float32),
            jax.ShapeDtypeStruct((NC, NPAD, D), jnp.float32),
        ),
        mesh=plsc.VectorSubcoreMesh(core_axis_name="c", subcore_axis_name="s",
                                    num_cores=NC, num_subcores=NS),
        scratch_types=[
            pltpu.VMEM_SHARED((NPAD, D), jnp.float32),
            pltpu.VMEM((CHUNK, D), jnp.float32),
            pltpu.VMEM((CHUNK, D), jnp.float32),
            pltpu.VMEM((CHUNK,), jnp.int32),
            pltpu.VMEM((CHUNK,), jnp.int32),
            pltpu.SemaphoreType.DMA,
        ],
    )


def kernel(x, t_feature_edge, t_edge_index, Wx1, bx1, Wx2, bx2,
           We1, be1, Wl, bl, Wr, br, We2, be2, Wf, bf):
    x_lin_1, x_lin_2 = _head(x, Wx1, bx1, Wx2, bx2)
    feature1 = _edge_linear(t_feature_edge, We1, be1)
    src = t_edge_index[0]
    dst = t_edge_index[1]
    agg_p, cnt_p = _sc_edge_kernel()(feature1, src, dst, x_lin_1)
    return _tail(agg_p[0], agg_p[1], cnt_p[0], cnt_p[1], x_lin_1, x_lin_2,
                 Wl, bl, Wr, br, We2, be2, Wf, bf)


# speedup vs baseline: 2.9686x; 1.1717x over previous
import functools

import jax
import jax.numpy as jnp
from jax import lax
from jax.experimental import pallas as pl
from jax.experimental.pallas import tpu as pltpu
from jax.experimental.pallas import tpu_sc as plsc

N = 10000
E = 320000
D = 128
DE = 16

NC = 2
NS = 16
NW = NC * NS
EPW = E // NW
CHUNK = 40
NCHUNK = EPW // CHUNK
NPAD = 10240
RPW = NPAD // NS
RZ = 128


def _head_body(x_ref, w1_ref, b1_ref, w2_ref, b2_ref, o1_ref, o2_ref):
    xb = x_ref[...]
    o1_ref[...] = jnp.maximum(
        jnp.dot(xb, w1_ref[...], preferred_element_type=jnp.float32) + b1_ref[...], 0.0)
    o2_ref[...] = jnp.maximum(
        jnp.dot(xb, w2_ref[...], preferred_element_type=jnp.float32) + b2_ref[...], 0.0)


def _head(x, Wx1, bx1, Wx2, bx2):
    nb = 10
    blk = N // nb
    return pl.pallas_call(
        _head_body,
        grid=(nb,),
        in_specs=[
            pl.BlockSpec((blk, D), lambda i: (i, 0)),
            pl.BlockSpec((D, D), lambda i: (0, 0)),
            pl.BlockSpec((1, D), lambda i: (0, 0)),
            pl.BlockSpec((D, D), lambda i: (0, 0)),
            pl.BlockSpec((1, D), lambda i: (0, 0)),
        ],
        out_specs=[
            pl.BlockSpec((blk, D), lambda i: (i, 0)),
            pl.BlockSpec((blk, D), lambda i: (i, 0)),
        ],
        out_shape=[
            jax.ShapeDtypeStruct((N, D), jnp.float32),
            jax.ShapeDtypeStruct((N, D), jnp.float32),
        ],
    )(x, Wx1, bx1.reshape(1, D), Wx2, bx2.reshape(1, D))


def _edge_body(fe_ref, w_ref, b_ref, o_ref):
    o_ref[...] = jnp.dot(fe_ref[...], w_ref[...],
                         preferred_element_type=jnp.float32) + b_ref[...]


def _edge_linear(t_feature_edge, We1, be1):
    nb = 80
    blk = E // nb
    return pl.pallas_call(
        _edge_body,
        grid=(nb,),
        in_specs=[
            pl.BlockSpec((blk, DE), lambda i: (i, 0)),
            pl.BlockSpec((DE, D), lambda i: (0, 0)),
            pl.BlockSpec((1, D), lambda i: (0, 0)),
        ],
        out_specs=pl.BlockSpec((blk, D), lambda i: (i, 0)),
        out_shape=jax.ShapeDtypeStruct((E, D), jnp.float32),
    )(t_feature_edge, We1, be1.reshape(1, D))


def _tail_body(a0_ref, a1_ref, c0_ref, c1_ref, x1_ref, x2_ref,
               wl_ref, bl_ref, wr_ref, br_ref, we2_ref, be2_ref,
               wf_ref, bf_ref, o_ref):
    deg = jnp.maximum(c0_ref[...][:, :1] + c1_ref[...][:, :1], 1.0)
    agg = (a0_ref[...] + a1_ref[...]) / deg
    t = (jnp.dot(agg, wl_ref[...], preferred_element_type=jnp.float32) + bl_ref[...]
         + jnp.dot(x1_ref[...], wr_ref[...], preferred_element_type=jnp.float32)
         + br_ref[...])
    h1 = jnp.maximum(
        jnp.dot(t, we2_ref[...], preferred_element_type=jnp.float32) + be2_ref[...], 0.0)
    o_ref[...] = (jnp.dot(h1 + x2_ref[...], wf_ref[...],
                          preferred_element_type=jnp.float32) + bf_ref[...])


def _tail(a0, a1, c0, c1, x1, x2, Wl, bl, Wr, br, We2, be2, Wf, bf):
    nb = 10
    blk = N // nb
    row = lambda i: (i, 0)
    full = lambda i: (0, 0)
    return pl.pallas_call(
        _tail_body,
        grid=(nb,),
        in_specs=[
            pl.BlockSpec((blk, D), row),
            pl.BlockSpec((blk, D), row),
            pl.BlockSpec((blk, D), row),
            pl.BlockSpec((blk, D), row),
            pl.BlockSpec((blk, D), row),
            pl.BlockSpec((blk, D), row),
            pl.BlockSpec((D, D), full),
            pl.BlockSpec((1, D), full),
            pl.BlockSpec((D, D), full),
            pl.BlockSpec((1, D), full),
            pl.BlockSpec((D, D), full),
            pl.BlockSpec((1, D), full),
            pl.BlockSpec((D, D), full),
            pl.BlockSpec((1, D), full),
        ],
        out_specs=pl.BlockSpec((blk, D), row),
        out_shape=jax.ShapeDtypeStruct((N, D), jnp.float32),
    )(a0, a1, c0, c1, x1, x2, Wl, bl.reshape(1, D), Wr, br.reshape(1, D),
      We2, be2.reshape(1, D), Wf, bf.reshape(1, D))


def _sc_body(f1_hbm, src_hbm, dst_hbm, x1_hbm, agg_out, cnt_out,
             agg_sh, f1_v, xg_v, src_v, dst_v, ldsem, gsem):
    cid = lax.axis_index("c")
    sid = lax.axis_index("s")
    wid = sid * NC + cid

    zero16 = jnp.zeros((16,), jnp.float32)
    one16 = jnp.ones((16,), jnp.float32)

    def fill_f1(p, val16):
        def fill(i, _):
            r = i // (D // 16)
            k = (i % (D // 16)) * 16
            f1_v[p, r, pl.ds(k, 16)] = val16
            return 0
        lax.fori_loop(0, CHUNK * (D // 16), fill, 0)

    def zero_acc():
        for b in range(RPW // CHUNK):
            r0 = sid * RPW + b * CHUNK
            pltpu.sync_copy(f1_v.at[0], agg_sh.at[pl.ds(r0, CHUNK)])

    def write_acc(out):
        for b in range(RPW // RZ):
            r0 = sid * RPW + b * RZ
            pltpu.sync_copy(agg_sh.at[pl.ds(r0, RZ)], out.at[cid, pl.ds(r0, RZ)])

    ebase = wid * EPW

    def issue_loads(t, p, full):
        b = ebase + t * CHUNK
        pltpu.async_copy(dst_hbm.at[pl.ds(b, CHUNK)], dst_v.at[p], ldsem)
        if full:
            pltpu.async_copy(src_hbm.at[pl.ds(b, CHUNK)], src_v.at[p], ldsem)
            pltpu.async_copy(f1_hbm.at[pl.ds(b, CHUNK)], f1_v.at[p], ldsem)

    def wait_loads(p, full):
        pltpu.make_async_copy(dst_hbm.at[pl.ds(0, CHUNK)], dst_v.at[p], ldsem).wait()
        if full:
            pltpu.make_async_copy(src_hbm.at[pl.ds(0, CHUNK)], src_v.at[p], ldsem).wait()
            pltpu.make_async_copy(f1_hbm.at[pl.ds(0, CHUNK)], f1_v.at[p], ldsem).wait()

    def edge_pass(ones_mode, out):
        full = not ones_mode

        def process(t, p, prefetch_t, guard):
            wait_loads(p, full)
            if full:
                gcp = pltpu.make_async_copy(x1_hbm.at[src_v.at[p]],
                                            xg_v.at[p], gsem)
                gcp.start()
            if guard is None:
                issue_loads(prefetch_t, 1 - p, full)
            else:
                @pl.when(guard)
                def _():
                    issue_loads(prefetch_t, 1 - p, full)
            if full:
                gcp.wait()

                def mul_body(r, _):
                    for k in range(D // 16):
                        f1_v[p, r, pl.ds(k * 16, 16)] = (
                            f1_v[p, r, pl.ds(k * 16, 16)]
                            * xg_v[p, r, pl.ds(k * 16, 16)])
                    return 0
                lax.fori_loop(0, CHUNK, mul_body, 0)
                pltpu.sync_copy(f1_v.at[p], agg_sh.at[dst_v.at[p]], add=True)
            else:
                pltpu.sync_copy(f1_v.at[0], agg_sh.at[dst_v.at[p]], add=True)

        issue_loads(0, 0, full)

        def pair_body(g, _):
            t0 = 2 * g
            process(t0, 0, t0 + 1, None)
            process(t0 + 1, 1, t0 + 2, t0 + 2 < NCHUNK)
            return 0
        lax.fori_loop(0, NCHUNK // 2, pair_body, 0)

        plsc.subcore_barrier()
        write_acc(out)
        plsc.subcore_barrier()

    fill_f1(0, zero16)
    zero_acc()
    plsc.subcore_barrier()
    edge_pass(False, agg_out)

    fill_f1(0, zero16)
    zero_acc()
    fill_f1(0, one16)
    plsc.subcore_barrier()
    edge_pass(True, cnt_out)


@functools.cache
def _sc_edge_kernel():
    return pl.kernel(
        _sc_body,
        out_type=(
            jax.ShapeDtypeStruct((NC, NPAD, D), jnp.float32),
            jax.ShapeDtypeStruct((NC, NPAD, D), jnp.float32),
        ),
        mesh=plsc.VectorSubcoreMesh(core_axis_name="c", subcore_axis_name="s",
                                    num_cores=NC, num_subcores=NS),
        scratch_types=[
            pltpu.VMEM_SHARED((NPAD, D), jnp.float32),
            pltpu.VMEM((2, CHUNK, D), jnp.float32),
            pltpu.VMEM((2, CHUNK, D), jnp.float32),
            pltpu.VMEM((2, CHUNK), jnp.int32),
            pltpu.VMEM((2, CHUNK), jnp.int32),
            pltpu.SemaphoreType.DMA,
            pltpu.SemaphoreType.DMA,
        ],
    )


def kernel(x, t_feature_edge, t_edge_index, Wx1, bx1, Wx2, bx2,
           We1, be1, Wl, bl, Wr, br, We2, be2, Wf, bf):
    x_lin_1, x_lin_2 = _head(x, Wx1, bx1, Wx2, bx2)
    feature1 = _edge_linear(t_feature_edge, We1, be1)
    src = t_edge_index[0]
    dst = t_edge_index[1]
    agg_p, cnt_p = _sc_edge_kernel()(feature1, src, dst, x_lin_1)
    return _tail(agg_p[0], agg_p[1], cnt_p[0], cnt_p[1], x_lin_1, x_lin_2,
                 Wl, bl, Wr, br, We2, be2, Wf, bf)


# speedup vs baseline: 3.3343x; 1.1232x over previous
import functools

import jax
import jax.numpy as jnp
from jax import lax
from jax.experimental import pallas as pl
from jax.experimental.pallas import tpu as pltpu
from jax.experimental.pallas import tpu_sc as plsc

N = 10000
E = 320000
D = 128
DE = 16

NC = 2
NS = 16
NW = NC * NS
EPW = E // NW
CHUNK = 40
NCHUNK = EPW // CHUNK
NPAD = 10240
RPW = NPAD // NS
RZ = 128


def _head_body(x_ref, w1_ref, b1_ref, w2_ref, b2_ref, o1_ref, o2_ref):
    xb = x_ref[...]
    o1_ref[...] = jnp.maximum(
        jnp.dot(xb, w1_ref[...], preferred_element_type=jnp.float32) + b1_ref[...], 0.0)
    o2_ref[...] = jnp.maximum(
        jnp.dot(xb, w2_ref[...], preferred_element_type=jnp.float32) + b2_ref[...], 0.0)


def _head(x, Wx1, bx1, Wx2, bx2):
    nb = 10
    blk = N // nb
    return pl.pallas_call(
        _head_body,
        grid=(nb,),
        in_specs=[
            pl.BlockSpec((blk, D), lambda i: (i, 0)),
            pl.BlockSpec((D, D), lambda i: (0, 0)),
            pl.BlockSpec((1, D), lambda i: (0, 0)),
            pl.BlockSpec((D, D), lambda i: (0, 0)),
            pl.BlockSpec((1, D), lambda i: (0, 0)),
        ],
        out_specs=[
            pl.BlockSpec((blk, D), lambda i: (i, 0)),
            pl.BlockSpec((blk, D), lambda i: (i, 0)),
        ],
        out_shape=[
            jax.ShapeDtypeStruct((N, D), jnp.float32),
            jax.ShapeDtypeStruct((N, D), jnp.float32),
        ],
    )(x, Wx1, bx1.reshape(1, D), Wx2, bx2.reshape(1, D))


def _edge_body(fe_ref, w_ref, b_ref, o_ref):
    o_ref[...] = jnp.dot(fe_ref[...], w_ref[...],
                         preferred_element_type=jnp.float32) + b_ref[...]


def _edge_linear(t_feature_edge, We1, be1):
    nb = 40
    blk = E // nb
    return pl.pallas_call(
        _edge_body,
        grid=(nb,),
        in_specs=[
            pl.BlockSpec((blk, DE), lambda i: (i, 0)),
            pl.BlockSpec((DE, D), lambda i: (0, 0)),
            pl.BlockSpec((1, D), lambda i: (0, 0)),
        ],
        out_specs=pl.BlockSpec((blk, D), lambda i: (i, 0)),
        out_shape=jax.ShapeDtypeStruct((E, D), jnp.float32),
    )(t_feature_edge, We1, be1.reshape(1, D))


def _tail_body(a0_ref, a1_ref, c0_ref, c1_ref, x1_ref, x2_ref,
               wl_ref, bl_ref, wr_ref, br_ref, we2_ref, be2_ref,
               wf_ref, bf_ref, o_ref):
    deg = jnp.maximum(c0_ref[...][:, :1] + c1_ref[...][:, :1], 1.0)
    agg = (a0_ref[...] + a1_ref[...]) / deg
    t = (jnp.dot(agg, wl_ref[...], preferred_element_type=jnp.float32) + bl_ref[...]
         + jnp.dot(x1_ref[...], wr_ref[...], preferred_element_type=jnp.float32)
         + br_ref[...])
    h1 = jnp.maximum(
        jnp.dot(t, we2_ref[...], preferred_element_type=jnp.float32) + be2_ref[...], 0.0)
    o_ref[...] = (jnp.dot(h1 + x2_ref[...], wf_ref[...],
                          preferred_element_type=jnp.float32) + bf_ref[...])


def _tail(a0, a1, c0, c1, x1, x2, Wl, bl, Wr, br, We2, be2, Wf, bf):
    nb = 10
    blk = N // nb
    row = lambda i: (i, 0)
    full = lambda i: (0, 0)
    return pl.pallas_call(
        _tail_body,
        grid=(nb,),
        in_specs=[
            pl.BlockSpec((blk, D), row),
            pl.BlockSpec((blk, D), row),
            pl.BlockSpec((blk, D), row),
            pl.BlockSpec((blk, D), row),
            pl.BlockSpec((blk, D), row),
            pl.BlockSpec((blk, D), row),
            pl.BlockSpec((D, D), full),
            pl.BlockSpec((1, D), full),
            pl.BlockSpec((D, D), full),
            pl.BlockSpec((1, D), full),
            pl.BlockSpec((D, D), full),
            pl.BlockSpec((1, D), full),
            pl.BlockSpec((D, D), full),
            pl.BlockSpec((1, D), full),
        ],
        out_specs=pl.BlockSpec((blk, D), row),
        out_shape=jax.ShapeDtypeStruct((N, D), jnp.float32),
    )(a0, a1, c0, c1, x1, x2, Wl, bl.reshape(1, D), Wr, br.reshape(1, D),
      We2, be2.reshape(1, D), Wf, bf.reshape(1, D))


def _sc_body(f1_hbm, src_hbm, dst_hbm, x1_hbm, agg_out, cnt_out,
             agg_sh, f1_v, xg_v, src_v, dst_v, ldsem, gsem, ssem):
    cid = lax.axis_index("c")
    sid = lax.axis_index("s")
    wid = sid * NC + cid

    zero16 = jnp.zeros((16,), jnp.float32)
    one16 = jnp.ones((16,), jnp.float32)

    def fill_f1(p, val16):
        def fill(i, _):
            r = i // (D // 16)
            k = (i % (D // 16)) * 16
            f1_v[p, r, pl.ds(k, 16)] = val16
            return 0
        lax.fori_loop(0, CHUNK * (D // 16), fill, 0)

    def zero_acc():
        for b in range(RPW // CHUNK):
            r0 = sid * RPW + b * CHUNK
            pltpu.sync_copy(f1_v.at[0], agg_sh.at[pl.ds(r0, CHUNK)])

    def write_acc(out):
        for b in range(RPW // RZ):
            r0 = sid * RPW + b * RZ
            pltpu.sync_copy(agg_sh.at[pl.ds(r0, RZ)], out.at[cid, pl.ds(r0, RZ)])

    ebase = wid * EPW

    def issue_loads(t, p, full):
        b = ebase + t * CHUNK
        pltpu.async_copy(dst_hbm.at[pl.ds(b, CHUNK)], dst_v.at[p], ldsem)
        if full:
            pltpu.async_copy(src_hbm.at[pl.ds(b, CHUNK)], src_v.at[p], ldsem)
            pltpu.async_copy(f1_hbm.at[pl.ds(b, CHUNK)], f1_v.at[p], ldsem)

    def wait_loads(p, full):
        pltpu.make_async_copy(dst_hbm.at[pl.ds(0, CHUNK)], dst_v.at[p], ldsem).wait()
        if full:
            pltpu.make_async_copy(src_hbm.at[pl.ds(0, CHUNK)], src_v.at[p], ldsem).wait()
            pltpu.make_async_copy(f1_hbm.at[pl.ds(0, CHUNK)], f1_v.at[p], ldsem).wait()

    def edge_pass(ones_mode, out):
        full = not ones_mode

        def wait_scatter(p):
            sp = p if full else 0
            pltpu.make_async_copy(f1_v.at[sp], agg_sh.at[dst_v.at[p]],
                                  ssem).wait()

        def process(t, p, prefetch_t, guard, swait):
            wait_loads(p, full)
            if full:
                gcp = pltpu.make_async_copy(x1_hbm.at[src_v.at[p]],
                                            xg_v.at[p], gsem)
                gcp.start()
            if swait is None:
                wait_scatter(1 - p)
            else:
                @pl.when(swait)
                def _():
                    wait_scatter(1 - p)
            if guard is None:
                issue_loads(prefetch_t, 1 - p, full)
            else:
                @pl.when(guard)
                def _():
                    issue_loads(prefetch_t, 1 - p, full)
            if full:
                gcp.wait()

                def mul_body(r, _):
                    for k in range(D // 16):
                        f1_v[p, r, pl.ds(k * 16, 16)] = (
                            f1_v[p, r, pl.ds(k * 16, 16)]
                            * xg_v[p, r, pl.ds(k * 16, 16)])
                    return 0
                lax.fori_loop(0, CHUNK, mul_body, 0)
                pltpu.async_copy(f1_v.at[p], agg_sh.at[dst_v.at[p]], ssem,
                                 add=True)
            else:
                pltpu.async_copy(f1_v.at[0], agg_sh.at[dst_v.at[p]], ssem,
                                 add=True)

        issue_loads(0, 0, full)

        def pair_body(g, _):
            t0 = 2 * g
            process(t0, 0, t0 + 1, None, g > 0)
            process(t0 + 1, 1, t0 + 2, t0 + 2 < NCHUNK, None)
            return 0
        lax.fori_loop(0, NCHUNK // 2, pair_body, 0)
        wait_scatter(1)

        plsc.subcore_barrier()
        write_acc(out)
        plsc.subcore_barrier()

    fill_f1(0, zero16)
    zero_acc()
    plsc.subcore_barrier()
    edge_pass(False, agg_out)

    fill_f1(0, zero16)
    zero_acc()
    fill_f1(0, one16)
    plsc.subcore_barrier()
    edge_pass(True, cnt_out)


@functools.cache
def _sc_edge_kernel():
    return pl.kernel(
        _sc_body,
        out_type=(
            jax.ShapeDtypeStruct((NC, NPAD, D), jnp.float32),
            jax.ShapeDtypeStruct((NC, NPAD, D), jnp.float32),
        ),
        mesh=plsc.VectorSubcoreMesh(core_axis_name="c", subcore_axis_name="s",
                                    num_cores=NC, num_subcores=NS),
        scratch_types=[
            pltpu.VMEM_SHARED((NPAD, D), jnp.float32),
            pltpu.VMEM((2, CHUNK, D), jnp.float32),
            pltpu.VMEM((2, CHUNK, D), jnp.float32),
            pltpu.VMEM((2, CHUNK), jnp.int32),
            pltpu.VMEM((2, CHUNK), jnp.int32),
            pltpu.SemaphoreType.DMA,
            pltpu.SemaphoreType.DMA,
            pltpu.SemaphoreType.DMA,
        ],
    )


def kernel(x, t_feature_edge, t_edge_index, Wx1, bx1, Wx2, bx2,
           We1, be1, Wl, bl, Wr, br, We2, be2, Wf, bf):
    x_lin_1, x_lin_2 = _head(x, Wx1, bx1, Wx2, bx2)
    feature1 = _edge_linear(t_feature_edge, We1, be1)
    src = t_edge_index[0]
    dst = t_edge_index[1]
    agg_p, cnt_p = _sc_edge_kernel()(feature1, src, dst, x_lin_1)
    return _tail(agg_p[0], agg_p[1], cnt_p[0], cnt_p[1], x_lin_1, x_lin_2,
                 Wl, bl, Wr, br, We2, be2, Wf, bf)


# speedup vs baseline: 3.9210x; 1.1760x over previous
import functools

import jax
import jax.numpy as jnp
from jax import lax
from jax.experimental import pallas as pl
from jax.experimental.pallas import tpu as pltpu
from jax.experimental.pallas import tpu_sc as plsc

N = 10000
E = 320000
D = 128
DE = 16

NC = 2
NS = 16
NW = NC * NS
EPW = E // NW
CHUNK = 40
NCHUNK = EPW // CHUNK
NPAD = 10240
RPW = NPAD // NS
RZ = 128


def _head_body(x_ref, w1_ref, b1_ref, w2_ref, b2_ref, o1_ref, o2_ref):
    xb = x_ref[...]
    o1_ref[...] = jnp.maximum(
        jnp.dot(xb, w1_ref[...], preferred_element_type=jnp.float32) + b1_ref[...], 0.0)
    o2_ref[...] = jnp.maximum(
        jnp.dot(xb, w2_ref[...], preferred_element_type=jnp.float32) + b2_ref[...], 0.0)


def _head(x, Wx1, bx1, Wx2, bx2):
    nb = 10
    blk = N // nb
    return pl.pallas_call(
        _head_body,
        grid=(nb,),
        in_specs=[
            pl.BlockSpec((blk, D), lambda i: (i, 0)),
            pl.BlockSpec((D, D), lambda i: (0, 0)),
            pl.BlockSpec((1, D), lambda i: (0, 0)),
            pl.BlockSpec((D, D), lambda i: (0, 0)),
            pl.BlockSpec((1, D), lambda i: (0, 0)),
        ],
        out_specs=[
            pl.BlockSpec((blk, D), lambda i: (i, 0)),
            pl.BlockSpec((blk, D), lambda i: (i, 0)),
        ],
        out_shape=[
            jax.ShapeDtypeStruct((N, D), jnp.float32),
            jax.ShapeDtypeStruct((N, D), jnp.float32),
        ],
    )(x, Wx1, bx1.reshape(1, D), Wx2, bx2.reshape(1, D))


def _edge_body(fe_ref, w_ref, b_ref, o_ref):
    o_ref[...] = lax.dot_general(
        fe_ref[...], w_ref[...], (((0,), (0,)), ((), ())),
        preferred_element_type=jnp.float32) + b_ref[...]


def _edge_linear(fet, We1, be1):
    nb = 50
    blk = E // nb
    return pl.pallas_call(
        _edge_body,
        grid=(nb,),
        in_specs=[
            pl.BlockSpec((DE, blk), lambda i: (0, i)),
            pl.BlockSpec((DE, D), lambda i: (0, 0)),
            pl.BlockSpec((1, D), lambda i: (0, 0)),
        ],
        out_specs=pl.BlockSpec((blk, D), lambda i: (i, 0)),
        out_shape=jax.ShapeDtypeStruct((E, D), jnp.float32),
    )(fet, We1, be1.reshape(1, D))


def _tail_body(a0_ref, a1_ref, c0_ref, c1_ref, x1_ref, x2_ref,
               wl_ref, bl_ref, wr_ref, br_ref, we2_ref, be2_ref,
               wf_ref, bf_ref, o_ref):
    deg = jnp.maximum(c0_ref[...][:, :1] + c1_ref[...][:, :1], 1.0)
    agg = (a0_ref[...] + a1_ref[...]) / deg
    t = (jnp.dot(agg, wl_ref[...], preferred_element_type=jnp.float32) + bl_ref[...]
         + jnp.dot(x1_ref[...], wr_ref[...], preferred_element_type=jnp.float32)
         + br_ref[...])
    h1 = jnp.maximum(
        jnp.dot(t, we2_ref[...], preferred_element_type=jnp.float32) + be2_ref[...], 0.0)
    o_ref[...] = (jnp.dot(h1 + x2_ref[...], wf_ref[...],
                          preferred_element_type=jnp.float32) + bf_ref[...])


def _tail(a0, a1, c0, c1, x1, x2, Wl, bl, Wr, br, We2, be2, Wf, bf):
    nb = 10
    blk = N // nb
    row = lambda i: (i, 0)
    full = lambda i: (0, 0)
    return pl.pallas_call(
        _tail_body,
        grid=(nb,),
        in_specs=[
            pl.BlockSpec((blk, D), row),
            pl.BlockSpec((blk, D), row),
            pl.BlockSpec((blk, D), row),
            pl.BlockSpec((blk, D), row),
            pl.BlockSpec((blk, D), row),
            pl.BlockSpec((blk, D), row),
            pl.BlockSpec((D, D), full),
            pl.BlockSpec((1, D), full),
            pl.BlockSpec((D, D), full),
            pl.BlockSpec((1, D), full),
            pl.BlockSpec((D, D), full),
            pl.BlockSpec((1, D), full),
            pl.BlockSpec((D, D), full),
            pl.BlockSpec((1, D), full),
        ],
        out_specs=pl.BlockSpec((blk, D), row),
        out_shape=jax.ShapeDtypeStruct((N, D), jnp.float32),
    )(a0, a1, c0, c1, x1, x2, Wl, bl.reshape(1, D), Wr, br.reshape(1, D),
      We2, be2.reshape(1, D), Wf, bf.reshape(1, D))


def _sc_body(f1_hbm, src_hbm, dst_hbm, x1_hbm, agg_out, cnt_out,
             agg_sh, f1_v, xg_v, src_v, dst_v, ldsem, gsem, ssem):
    cid = lax.axis_index("c")
    sid = lax.axis_index("s")
    wid = sid * NC + cid

    zero16 = jnp.zeros((16,), jnp.float32)
    one16 = jnp.ones((16,), jnp.float32)

    def fill_f1(p, val16):
        def fill(i, _):
            r = i // (D // 16)
            k = (i % (D // 16)) * 16
            f1_v[p, r, pl.ds(k, 16)] = val16
            return 0
        lax.fori_loop(0, CHUNK * (D // 16), fill, 0)

    def zero_acc():
        for b in range(RPW // CHUNK):
            r0 = sid * RPW + b * CHUNK
            pltpu.sync_copy(f1_v.at[0], agg_sh.at[pl.ds(r0, CHUNK)])

    def write_acc(out):
        for b in range(RPW // RZ):
            r0 = sid * RPW + b * RZ
            pltpu.sync_copy(agg_sh.at[pl.ds(r0, RZ)], out.at[cid, pl.ds(r0, RZ)])

    ebase = wid * EPW

    def issue_loads(t, p, full):
        b = ebase + t * CHUNK
        pltpu.async_copy(dst_hbm.at[pl.ds(b, CHUNK)], dst_v.at[p], ldsem)
        if full:
            pltpu.async_copy(src_hbm.at[pl.ds(b, CHUNK)], src_v.at[p], ldsem)
            pltpu.async_copy(f1_hbm.at[pl.ds(b, CHUNK)], f1_v.at[p], ldsem)

    def wait_loads(p, full):
        pltpu.make_async_copy(dst_hbm.at[pl.ds(0, CHUNK)], dst_v.at[p], ldsem).wait()
        if full:
            pltpu.make_async_copy(src_hbm.at[pl.ds(0, CHUNK)], src_v.at[p], ldsem).wait()
            pltpu.make_async_copy(f1_hbm.at[pl.ds(0, CHUNK)], f1_v.at[p], ldsem).wait()

    def edge_pass(ones_mode, out):
        full = not ones_mode

        def wait_scatter(p):
            sp = p if full else 0
            pltpu.make_async_copy(f1_v.at[sp], agg_sh.at[dst_v.at[p]],
                                  ssem).wait()

        def process(t, p, prefetch_t, guard, swait):
            wait_loads(p, full)
            if full:
                gcp = pltpu.make_async_copy(x1_hbm.at[src_v.at[p]],
                                            xg_v.at[p], gsem)
                gcp.start()
            if swait is None:
                wait_scatter(1 - p)
            else:
                @pl.when(swait)
                def _():
                    wait_scatter(1 - p)
            if guard is None:
                issue_loads(prefetch_t, 1 - p, full)
            else:
                @pl.when(guard)
                def _():
                    issue_loads(prefetch_t, 1 - p, full)
            if full:
                gcp.wait()

                def mul_body(r, _):
                    for k in range(D // 16):
                        f1_v[p, r, pl.ds(k * 16, 16)] = (
                            f1_v[p, r, pl.ds(k * 16, 16)]
                            * xg_v[p, r, pl.ds(k * 16, 16)])
                    return 0
                lax.fori_loop(0, CHUNK, mul_body, 0)
                pltpu.async_copy(f1_v.at[p], agg_sh.at[dst_v.at[p]], ssem,
                                 add=True)
            else:
                pltpu.async_copy(f1_v.at[0], agg_sh.at[dst_v.at[p]], ssem,
                                 add=True)

        issue_loads(0, 0, full)

        def pair_body(g, _):
            t0 = 2 * g
            process(t0, 0, t0 + 1, None, g > 0)
            process(t0 + 1, 1, t0 + 2, t0 + 2 < NCHUNK, None)
            return 0
        lax.fori_loop(0, NCHUNK // 2, pair_body, 0)
        wait_scatter(1)

        plsc.subcore_barrier()
        write_acc(out)
        plsc.subcore_barrier()

    fill_f1(0, zero16)
    zero_acc()
    plsc.subcore_barrier()
    edge_pass(False, agg_out)

    fill_f1(0, zero16)
    zero_acc()
    fill_f1(0, one16)
    plsc.subcore_barrier()
    edge_pass(True, cnt_out)


@functools.cache
def _sc_edge_kernel():
    return pl.kernel(
        _sc_body,
        out_type=(
            jax.ShapeDtypeStruct((NC, NPAD, D), jnp.float32),
            jax.ShapeDtypeStruct((NC, NPAD, D), jnp.float32),
        ),
        mesh=plsc.VectorSubcoreMesh(core_axis_name="c", subcore_axis_name="s",
                                    num_cores=NC, num_subcores=NS),
        scratch_types=[
            pltpu.VMEM_SHARED((NPAD, D), jnp.float32),
            pltpu.VMEM((2, CHUNK, D), jnp.float32),
            pltpu.VMEM((2, CHUNK, D), jnp.float32),
            pltpu.VMEM((2, CHUNK), jnp.int32),
            pltpu.VMEM((2, CHUNK), jnp.int32),
            pltpu.SemaphoreType.DMA,
            pltpu.SemaphoreType.DMA,
            pltpu.SemaphoreType.DMA,
        ],
    )


def kernel(x, t_feature_edge, t_edge_index, Wx1, bx1, Wx2, bx2,
           We1, be1, Wl, bl, Wr, br, We2, be2, Wf, bf):
    x_lin_1, x_lin_2 = _head(x, Wx1, bx1, Wx2, bx2)
    feature1 = _edge_linear(t_feature_edge.T, We1, be1)
    src = t_edge_index[0]
    dst = t_edge_index[1]
    agg_p, cnt_p = _sc_edge_kernel()(feature1, src, dst, x_lin_1)
    return _tail(agg_p[0], agg_p[1], cnt_p[0], cnt_p[1], x_lin_1, x_lin_2,
                 Wl, bl, Wr, br, We2, be2, Wf, bf)
